# Initial kernel scaffold; baseline (speedup 1.0000x reference)
import jax
import jax.numpy as jnp
from jax.experimental import pallas as pl


def kernel(cls_out0, cls_out1, cls_out2, reg_out0, reg_out1, reg_out2, obj_out0, obj_out1, obj_out2, images_hw):
    raise NotImplementedError("write your pallas kernel here")


# speedup vs baseline: 25.0817x; 25.0817x over previous
import functools

import jax
import jax.numpy as jnp
from jax.experimental import pallas as pl
from jax.experimental.pallas import tpu as pltpu

B = 16
NUM_CLASSES = 80
FEAT_SIZES = ((80, 80), (40, 40), (20, 20))
STRIDES = (8, 16, 32)
NMS_THRESHOLD = 0.65
SCORE_THR = 0.01
PRE_NMS_K = 2000
MAX_DETS = 100
CLASS_OFFSET = 8192.0

N_ANCH = sum(h * w for h, w in FEAT_SIZES)
A = 8448
ONE_BITS = 0x3F800000


def _grid_priors_padded():
    pts = []
    for (h, w), s in zip(FEAT_SIZES, STRIDES):
        ys, xs = jnp.meshgrid(
            jnp.arange(h, dtype=jnp.float32) * s,
            jnp.arange(w, dtype=jnp.float32) * s,
            indexing="ij",
        )
        stride = jnp.full((h * w,), float(s), dtype=jnp.float32)
        pts.append(jnp.stack([xs.reshape(-1), ys.reshape(-1), stride, stride], axis=-1))
    p = jnp.concatenate(pts, axis=0)
    p = jnp.concatenate(
        [p, jnp.concatenate([jnp.zeros((A - N_ANCH, 2), jnp.float32),
                             jnp.ones((A - N_ANCH, 2), jnp.float32)], axis=1)],
        axis=0,
    )
    return p.T


def _prep_kernel(cls_ref, reg_ref, obj_ref, pts_ref, meta_ref):
    cls = cls_ref[0]
    sig = jax.nn.sigmoid(cls)
    m = jnp.max(sig, axis=0, keepdims=True)
    cidx = jax.lax.broadcasted_iota(jnp.int32, sig.shape, 0)
    lab = jnp.min(jnp.where(sig == m, cidx, NUM_CLASSES), axis=0,
                  keepdims=True).astype(jnp.float32)
    obj = jax.nn.sigmoid(obj_ref[0])
    score = m * obj
    score = jnp.where(score >= SCORE_THR, score, -1.0)

    px = pts_ref[0:1, :]
    py = pts_ref[1:2, :]
    ps = pts_ref[2:3, :]
    rx = reg_ref[0, 0:1, :]
    ry = reg_ref[0, 1:2, :]
    rw = reg_ref[0, 2:3, :]
    rh = reg_ref[0, 3:4, :]
    cx = rx * ps + px
    cy = ry * ps + py
    w = jnp.exp(rw) * ps
    h = jnp.exp(rh) * ps
    x1 = cx - w / 2.0
    y1 = cy - h / 2.0
    x2 = cx + w / 2.0
    y2 = cy + h / 2.0
    off = lab * CLASS_OFFSET
    meta_ref[0] = jnp.concatenate(
        [x1, y1, x2, y2, x1 + off, y1 + off, x2 + off, y2 + off, score, lab],
        axis=0,
    )


def _nms_kernel(meta_ref, out_ref, swork_ref, area2_ref):
    s = meta_ref[:, 8, :]
    bits = jax.lax.bitcast_convert_type(s, jnp.int32)
    nvalid = jnp.sum((s >= 0.0).astype(jnp.int32), axis=1, keepdims=True)

    def bis_body(_, lohi):
        lo, hi = lohi
        mid = (lo + hi) >> 1
        cnt = jnp.sum((bits >= mid).astype(jnp.int32), axis=1, keepdims=True)
        ge = cnt >= PRE_NMS_K
        return jnp.where(ge, mid, lo), jnp.where(ge, hi, mid)

    lo0 = jnp.zeros((B, 1), jnp.int32)
    hi0 = jnp.full((B, 1), ONE_BITS, jnp.int32)
    lo, hi = jax.lax.fori_loop(0, 31, bis_body, (lo0, hi0))
    tbits = jnp.where(nvalid >= PRE_NMS_K, lo, 0)

    swork_ref[...] = jnp.where(bits >= tbits, s, -2.0)
    ox1 = meta_ref[:, 4, :]
    oy1 = meta_ref[:, 5, :]
    ox2 = meta_ref[:, 6, :]
    oy2 = meta_ref[:, 7, :]
    area2_ref[...] = jnp.clip(ox2 - ox1, 0.0) * jnp.clip(oy2 - oy1, 0.0)

    li = jax.lax.broadcasted_iota(jnp.int32, (B, A), 1)

    def nms_body(i, _):
        sw = swork_ref[...]
        m = jnp.max(sw, axis=1, keepdims=True)
        kept = m >= 0.0
        pos = jnp.min(jnp.where(sw == m, li, A), axis=1, keepdims=True)
        oh = li == pos

        def gather(row):
            return jnp.sum(jnp.where(oh, meta_ref[:, row, :], 0.0), axis=1,
                           keepdims=True)

        bx1, by1, bx2, by2 = gather(0), gather(1), gather(2), gather(3)
        qx1, qy1, qx2, qy2 = gather(4), gather(5), gather(6), gather(7)
        lab = gather(9)

        xx1 = jnp.maximum(qx1, meta_ref[:, 4, :])
        yy1 = jnp.maximum(qy1, meta_ref[:, 5, :])
        xx2 = jnp.minimum(qx2, meta_ref[:, 6, :])
        yy2 = jnp.minimum(qy2, meta_ref[:, 7, :])
        inter = jnp.clip(xx2 - xx1, 0.0) * jnp.clip(yy2 - yy1, 0.0)
        a1 = jnp.clip(qx2 - qx1, 0.0) * jnp.clip(qy2 - qy1, 0.0)
        iou = inter / (a1 + area2_ref[...] - inter + 1e-9)
        supp = (iou > NMS_THRESHOLD) | oh
        swork_ref[...] = jnp.where(kept & supp, -3.0, sw)

        row = jnp.concatenate(
            [jnp.where(kept, bx1, 0.0),
             jnp.where(kept, by1, 0.0),
             jnp.where(kept, bx2, 0.0),
             jnp.where(kept, by2, 0.0),
             jnp.where(kept, m, 0.0),
             jnp.where(kept, lab, -1.0),
             jnp.zeros((B, 2), jnp.float32)],
            axis=1,
        )
        out_ref[:, pl.ds(i, 1), :] = row[:, None, :]
        return 0

    jax.lax.fori_loop(0, MAX_DETS, nms_body, 0)


@functools.partial(jax.jit, static_argnames=())
def kernel(cls_out0, cls_out1, cls_out2, reg_out0, reg_out1, reg_out2,
           obj_out0, obj_out1, obj_out2, images_hw=None):
    cls_flat = jnp.concatenate(
        [x.reshape(B, NUM_CLASSES, -1) for x in (cls_out0, cls_out1, cls_out2)],
        axis=2)
    reg_flat = jnp.concatenate(
        [x.reshape(B, 4, -1) for x in (reg_out0, reg_out1, reg_out2)], axis=2)
    obj_flat = jnp.concatenate(
        [x.reshape(B, 1, -1) for x in (obj_out0, obj_out1, obj_out2)], axis=2)
    pad = A - N_ANCH
    cls_flat = jnp.pad(cls_flat, ((0, 0), (0, 0), (0, pad)))
    reg_flat = jnp.pad(reg_flat, ((0, 0), (0, 0), (0, pad)))
    obj_flat = jnp.pad(obj_flat, ((0, 0), (0, 0), (0, pad)),
                       constant_values=-30.0)
    pts = _grid_priors_padded()

    meta = pl.pallas_call(
        _prep_kernel,
        grid=(B,),
        in_specs=[
            pl.BlockSpec((1, NUM_CLASSES, A), lambda b: (b, 0, 0)),
            pl.BlockSpec((1, 4, A), lambda b: (b, 0, 0)),
            pl.BlockSpec((1, 1, A), lambda b: (b, 0, 0)),
            pl.BlockSpec((4, A), lambda b: (0, 0)),
        ],
        out_specs=pl.BlockSpec((1, 10, A), lambda b: (b, 0, 0)),
        out_shape=jax.ShapeDtypeStruct((B, 10, A), jnp.float32),
    )(cls_flat, reg_flat, obj_flat, pts)

    out = pl.pallas_call(
        _nms_kernel,
        in_specs=[pl.BlockSpec((B, 10, A), lambda: (0, 0, 0))],
        out_specs=pl.BlockSpec((B, MAX_DETS, 8), lambda: (0, 0, 0)),
        out_shape=jax.ShapeDtypeStruct((B, MAX_DETS, 8), jnp.float32),
        scratch_shapes=[
            pltpu.VMEM((B, A), jnp.float32),
            pltpu.VMEM((B, A), jnp.float32),
        ],
    )(meta)

    out_boxes = out[:, :, 0:4]
    out_scores = out[:, :, 4]
    out_labels = out[:, :, 5].astype(jnp.int32)
    return out_boxes, out_scores, out_labels


# speedup vs baseline: 29.6954x; 1.1839x over previous
import functools

import jax
import jax.numpy as jnp
from jax.experimental import pallas as pl
from jax.experimental.pallas import tpu as pltpu

B = 16
NUM_CLASSES = 80
FEAT_SIZES = ((80, 80), (40, 40), (20, 20))
STRIDES = (8, 16, 32)
NMS_THRESHOLD = 0.65
SCORE_THR = 0.01
PRE_NMS_K = 2000
MAX_DETS = 100
CLASS_OFFSET = 8192.0

N_ANCH = sum(h * w for h, w in FEAT_SIZES)
A = 8448
ONE_BITS = 0x3F800000


def _grid_priors_padded():
    pts = []
    for (h, w), s in zip(FEAT_SIZES, STRIDES):
        ys, xs = jnp.meshgrid(
            jnp.arange(h, dtype=jnp.float32) * s,
            jnp.arange(w, dtype=jnp.float32) * s,
            indexing="ij",
        )
        stride = jnp.full((h * w,), float(s), dtype=jnp.float32)
        pts.append(jnp.stack([xs.reshape(-1), ys.reshape(-1), stride, stride], axis=-1))
    p = jnp.concatenate(pts, axis=0)
    p = jnp.concatenate(
        [p, jnp.concatenate([jnp.zeros((A - N_ANCH, 2), jnp.float32),
                             jnp.ones((A - N_ANCH, 2), jnp.float32)], axis=1)],
        axis=0,
    )
    return p.T


def _prep_kernel(cls_ref, reg_ref, obj_ref, pts_ref, meta_ref):
    cls = cls_ref[0]
    sig = jax.nn.sigmoid(cls)
    m = jnp.max(sig, axis=0, keepdims=True)
    cidx = jax.lax.broadcasted_iota(jnp.int32, sig.shape, 0)
    lab = jnp.min(jnp.where(sig == m, cidx, NUM_CLASSES), axis=0,
                  keepdims=True).astype(jnp.float32)
    obj = jax.nn.sigmoid(obj_ref[0])
    score = m * obj
    score = jnp.where(score >= SCORE_THR, score, -1.0)

    px = pts_ref[0:1, :]
    py = pts_ref[1:2, :]
    ps = pts_ref[2:3, :]
    rx = reg_ref[0, 0:1, :]
    ry = reg_ref[0, 1:2, :]
    rw = reg_ref[0, 2:3, :]
    rh = reg_ref[0, 3:4, :]
    cx = rx * ps + px
    cy = ry * ps + py
    w = jnp.exp(rw) * ps
    h = jnp.exp(rh) * ps
    x1 = cx - w / 2.0
    y1 = cy - h / 2.0
    x2 = cx + w / 2.0
    y2 = cy + h / 2.0
    off = lab * CLASS_OFFSET
    meta_ref[0] = jnp.concatenate(
        [x1, y1, x2, y2, x1 + off, y1 + off, x2 + off, y2 + off, score, lab],
        axis=0,
    )


def _nms_kernel(meta_ref, out_ref, swork_ref, area2_ref):
    s = meta_ref[:, 8, :]
    bits = jax.lax.bitcast_convert_type(s, jnp.int32)
    nvalid = jnp.sum((s >= 0.0).astype(jnp.int32), axis=1, keepdims=True)

    def bis_body(_, lohi):
        lo, hi = lohi
        mid = (lo + hi) >> 1
        cnt = jnp.sum((bits >= mid).astype(jnp.int32), axis=1, keepdims=True)
        ge = cnt >= PRE_NMS_K
        return jnp.where(ge, mid, lo), jnp.where(ge, hi, mid)

    lo0 = jnp.zeros((B, 1), jnp.int32)
    hi0 = jnp.full((B, 1), ONE_BITS, jnp.int32)
    lo, hi = jax.lax.fori_loop(0, 31, bis_body, (lo0, hi0))
    tbits = jnp.where(nvalid >= PRE_NMS_K, lo, 0)

    swork_ref[...] = jnp.where(bits >= tbits, s, -2.0)
    ox1 = meta_ref[:, 4, :]
    oy1 = meta_ref[:, 5, :]
    ox2 = meta_ref[:, 6, :]
    oy2 = meta_ref[:, 7, :]
    area2_ref[...] = jnp.clip(ox2 - ox1, 0.0) * jnp.clip(oy2 - oy1, 0.0)

    li = jax.lax.broadcasted_iota(jnp.int32, (B, A), 1)

    def nms_body(i, _):
        sw = swork_ref[...]
        m = jnp.max(sw, axis=1, keepdims=True)
        kept = m >= 0.0
        pos = jnp.min(jnp.where(sw == m, li, A), axis=1, keepdims=True)
        oh = li == pos

        ohf = oh.astype(jnp.float32)

        def gather(row):
            return jnp.sum(ohf * meta_ref[:, row, :], axis=1,
                           keepdims=True)

        qx1, qy1, qx2, qy2 = gather(4), gather(5), gather(6), gather(7)
        lab = gather(9)
        off = lab * CLASS_OFFSET
        bx1, by1, bx2, by2 = qx1 - off, qy1 - off, qx2 - off, qy2 - off

        xx1 = jnp.maximum(qx1, meta_ref[:, 4, :])
        yy1 = jnp.maximum(qy1, meta_ref[:, 5, :])
        xx2 = jnp.minimum(qx2, meta_ref[:, 6, :])
        yy2 = jnp.minimum(qy2, meta_ref[:, 7, :])
        inter = jnp.clip(xx2 - xx1, 0.0) * jnp.clip(yy2 - yy1, 0.0)
        a1 = jnp.clip(qx2 - qx1, 0.0) * jnp.clip(qy2 - qy1, 0.0)
        iou = inter / (a1 + area2_ref[...] - inter + 1e-9)
        supp = (iou > NMS_THRESHOLD) | oh
        swork_ref[...] = jnp.where(kept & supp, -3.0, sw)

        row = jnp.concatenate(
            [jnp.where(kept, bx1, 0.0),
             jnp.where(kept, by1, 0.0),
             jnp.where(kept, bx2, 0.0),
             jnp.where(kept, by2, 0.0),
             jnp.where(kept, m, 0.0),
             jnp.where(kept, lab, -1.0),
             jnp.zeros((B, 2), jnp.float32)],
            axis=1,
        )
        out_ref[:, pl.ds(i, 1), :] = row[:, None, :]
        return 0

    jax.lax.fori_loop(0, MAX_DETS, nms_body, 0)


@functools.partial(jax.jit, static_argnames=())
def kernel(cls_out0, cls_out1, cls_out2, reg_out0, reg_out1, reg_out2,
           obj_out0, obj_out1, obj_out2, images_hw=None):
    cls_flat = jnp.concatenate(
        [x.reshape(B, NUM_CLASSES, -1) for x in (cls_out0, cls_out1, cls_out2)],
        axis=2)
    reg_flat = jnp.concatenate(
        [x.reshape(B, 4, -1) for x in (reg_out0, reg_out1, reg_out2)], axis=2)
    obj_flat = jnp.concatenate(
        [x.reshape(B, 1, -1) for x in (obj_out0, obj_out1, obj_out2)], axis=2)
    pad = A - N_ANCH
    cls_flat = jnp.pad(cls_flat, ((0, 0), (0, 0), (0, pad)))
    reg_flat = jnp.pad(reg_flat, ((0, 0), (0, 0), (0, pad)))
    obj_flat = jnp.pad(obj_flat, ((0, 0), (0, 0), (0, pad)),
                       constant_values=-30.0)
    pts = _grid_priors_padded()

    meta = pl.pallas_call(
        _prep_kernel,
        grid=(B,),
        in_specs=[
            pl.BlockSpec((1, NUM_CLASSES, A), lambda b: (b, 0, 0)),
            pl.BlockSpec((1, 4, A), lambda b: (b, 0, 0)),
            pl.BlockSpec((1, 1, A), lambda b: (b, 0, 0)),
            pl.BlockSpec((4, A), lambda b: (0, 0)),
        ],
        out_specs=pl.BlockSpec((1, 10, A), lambda b: (b, 0, 0)),
        out_shape=jax.ShapeDtypeStruct((B, 10, A), jnp.float32),
    )(cls_flat, reg_flat, obj_flat, pts)

    out = pl.pallas_call(
        _nms_kernel,
        in_specs=[pl.BlockSpec((B, 10, A), lambda: (0, 0, 0))],
        out_specs=pl.BlockSpec((B, MAX_DETS, 8), lambda: (0, 0, 0)),
        out_shape=jax.ShapeDtypeStruct((B, MAX_DETS, 8), jnp.float32),
        scratch_shapes=[
            pltpu.VMEM((B, A), jnp.float32),
            pltpu.VMEM((B, A), jnp.float32),
        ],
    )(meta)

    out_boxes = out[:, :, 0:4]
    out_scores = out[:, :, 4]
    out_labels = out[:, :, 5].astype(jnp.int32)
    return out_boxes, out_scores, out_labels
